# baseline (speedup 1.0000x reference)
import functools

import jax
import jax.numpy as jnp
from jax import lax
from jax.experimental import pallas as pl
from jax.experimental.pallas import tpu as pltpu
from jax.experimental.pallas import tpu_sc as plsc

_TOPK = 8


def _tc_body(xe_ref, xo_ref, main_ref, val_ref, idx_ref):
    p = pl.program_id(2)

    @pl.when(p == 0)
    def _compute():
        xe = xe_ref[0]
        xo = xo_ref[0]
        low2 = (xe + xo) * 0.5
        high = xe - xo
        T2, FB = high.shape

        hb = jax.lax.bitcast_convert_type(high, jnp.int32)
        mbits = (hb & jnp.int32(0x7FFFFFFE)) | ((hb >> 31) & jnp.int32(1))
        m = jax.lax.bitcast_convert_type(mbits, jnp.float32)

        iota = jax.lax.broadcasted_iota(jnp.int32, (T2, FB), 0)
        rows_v = []
        rows_i = []
        for _ in range(_TOPK):
            mx = jnp.max(m, axis=0, keepdims=True)
            eq = m >= mx
            cand = jnp.where(eq, iota, jnp.int32(T2))
            imin = jnp.min(cand, axis=0, keepdims=True)
            m = jnp.where(eq, jnp.float32(-1.0), m)
            rows_v.append(mx)
            rows_i.append(imin)
        vk = jnp.concatenate(rows_v, axis=0)
        ik = jnp.concatenate(rows_i, axis=0)
        vb = jax.lax.bitcast_convert_type(vk, jnp.int32)
        v = jax.lax.bitcast_convert_type(
            (vb & jnp.int32(0x7FFFFFFE)) | ((vb & jnp.int32(1)) << 31),
            jnp.float32)
        val_ref[0] = v * 0.5
        idx_ref[0] = ik
        main_ref[0] = low2

    @pl.when(p == 1)
    def _write_odd():
        main_ref[0] = (xe_ref[0] + xo_ref[0]) * 0.5


def _tc_stage(xr, B, T2, F, FB):
    NF = F // FB
    spec_e = pl.BlockSpec((1, T2, FB), lambda b, fb, p: (b, 0, fb))
    spec_o = pl.BlockSpec((1, T2, FB), lambda b, fb, p: (b, 0, NF + fb))
    spec_main = pl.BlockSpec((1, T2, FB), lambda b, fb, p: (b, 0, p * NF + fb))
    spec_topk = pl.BlockSpec((1, _TOPK, FB), lambda b, fb, p: (b, 0, fb))

    return pl.pallas_call(
        _tc_body,
        grid=(B, NF, 2),
        in_specs=[spec_e, spec_o],
        out_specs=[spec_main, spec_topk, spec_topk],
        out_shape=[
            jax.ShapeDtypeStruct((B, T2, 2 * F), jnp.float32),
            jax.ShapeDtypeStruct((B, _TOPK, F), jnp.float32),
            jax.ShapeDtypeStruct((B, _TOPK, F), jnp.int32),
        ],
    )(xr, xr)


def _make_sc_build_detail(B, T, F):
    N = B * T * F
    NW = 32
    REG = N // NW
    CHUNK = 16384
    NDMA = REG // CHUNK
    ENT = B * _TOPK * F // NW
    ROWS = ENT // 128
    mesh = plsc.VectorSubcoreMesh(core_axis_name="c", subcore_axis_name="s")

    @functools.partial(
        pl.kernel,
        out_type=jax.ShapeDtypeStruct((N,), jnp.float32),
        mesh=mesh,
        scratch_types=[
            pltpu.VMEM((CHUNK,), jnp.float32),
            pltpu.VMEM((ROWS, 128), jnp.float32),
            pltpu.VMEM((ROWS, 128), jnp.float32),
            pltpu.VMEM((ROWS, 128), jnp.int32),
            pltpu.VMEM((ROWS, 128), jnp.int32),
            pltpu.VMEM((ROWS, 128), jnp.int32),
            pltpu.SemaphoreType.DMA,
            pltpu.SemaphoreType.DMA,
        ],
    )
    def sc_build_detail(val_hbm, idx_hbm, det_hbm, zbuf, vbuf, nbuf, tbuf,
                        oebuf, oobuf, sem0, sem1):
        c = lax.axis_index("c")
        s = lax.axis_index("s")
        wid = c * 16 + s

        cp_v = pltpu.async_copy(val_hbm.at[pl.ds(wid * ROWS, ROWS)], vbuf, sem1)
        cp_t = pltpu.async_copy(idx_hbm.at[pl.ds(wid * ROWS, ROWS)], tbuf, sem1)

        def _zero(i, carry):
            zbuf[pl.ds(i * 16, 16)] = jnp.zeros((16,), jnp.float32)
            return carry
        lax.fori_loop(0, CHUNK // 16, _zero, 0)

        hs = [
            pltpu.async_copy(
                zbuf, det_hbm.at[pl.ds(wid * REG + j * CHUNK, CHUNK)], sem0)
            for j in range(NDMA)
        ]

        cp_v.wait()
        cp_t.wait()
        b = wid // _TOPK
        for j in range(ROWS):
            for k in range(8):
                sl = pl.ds(k * 16, 16)
                t = tbuf[j, sl]
                v = vbuf[j, sl]
                f = jnp.int32(j * 128 + k * 16) + lax.iota(jnp.int32, 16)
                oe = b * jnp.int32(T * F) + t * jnp.int32(2 * F) + f
                oebuf[j, sl] = oe
                oobuf[j, sl] = oe + jnp.int32(F)
                nbuf[j, sl] = -v

        for h in hs:
            h.wait()
        plsc.subcore_barrier()

        sc_hs = []
        for j in range(ROWS):
            sc_hs.append(
                pltpu.async_copy(vbuf.at[j], det_hbm.at[oebuf.at[j]], sem1))
            sc_hs.append(
                pltpu.async_copy(nbuf.at[j], det_hbm.at[oobuf.at[j]], sem1))
        for h in sc_hs:
            h.wait()

    return sc_build_detail


def kernel(x):
    B, T, F = x.shape
    T2 = T // 2
    FB = min(128, F)
    xr = x.reshape(B, T2, 2 * F)

    main_r, val8, idx8 = _tc_stage(xr, B, T2, F, FB)

    del val8, idx8
    return main_r.reshape(B, T, F), main_r.reshape(B, T, F)


# speedup vs baseline: 1.2460x; 1.2460x over previous
import jax
import jax.numpy as jnp
from jax.experimental import pallas as pl
from jax.experimental.pallas import tpu as pltpu

_TOPK = 8


def _oemerge(a, b):
    if len(a) == 1:
        return [jnp.maximum(a[0], b[0]), jnp.minimum(a[0], b[0])]
    c = _oemerge(a[0::2], b[0::2])
    d = _oemerge(a[1::2], b[1::2])
    out = [c[0]]
    for i in range(len(d) - 1):
        out.append(jnp.maximum(d[i], c[i + 1]))
        out.append(jnp.minimum(d[i], c[i + 1]))
    out.append(d[-1])
    return out


def _top8_merge(a, b):
    m = [jnp.maximum(a[i], b[7 - i]) for i in range(8)]
    for dist in (4, 2, 1):
        nm = list(m)
        for i in range(8):
            if (i & dist) == 0:
                nm[i] = jnp.maximum(m[i], m[i + dist])
                nm[i + dist] = jnp.minimum(m[i], m[i + dist])
        m = nm
    return m


def _tc_body(xe_ref, xo_ref, main_ref, det_ref, do_s):
    p = pl.program_id(2)

    @pl.when(p == 0)
    def _compute():
        xe = xe_ref[0]
        xo = xo_ref[0]
        low2 = (xe + xo) * 0.5
        high = xe - xo
        T2 = high.shape[0]

        cur = [[jnp.abs(high[i * 8:(i + 1) * 8, :])] for i in range(T2 // 8)]
        while len(cur) > 1:
            nxt = []
            for i in range(0, len(cur), 2):
                a, b = cur[i], cur[i + 1]
                if len(a) < 8:
                    nxt.append(_oemerge(a, b))
                else:
                    nxt.append(_top8_merge(a, b))
            cur = nxt
        S = jnp.concatenate(cur[0], axis=0)

        mx = None
        for _ in range(_TOPK):
            mx = jnp.max(S, axis=0, keepdims=True)
            S = jnp.where(S >= mx, jnp.float32(-1.0), S)
        thresh = mx

        det = jnp.where(jnp.abs(high) >= thresh, high * 0.5,
                        jnp.zeros_like(high))
        main_ref[0] = low2
        det_ref[0] = det
        do_s[...] = -det

    @pl.when(p == 1)
    def _write_odd():
        main_ref[0] = (xe_ref[0] + xo_ref[0]) * 0.5
        det_ref[0] = do_s[...]


def kernel(x):
    B, T, F = x.shape
    T2 = T // 2
    FB = min(256, F)
    NF = F // FB
    xr = x.reshape(B, T2, 2 * F)

    spec_e = pl.BlockSpec((1, T2, FB), lambda b, fb, p: (b, 0, fb))
    spec_o = pl.BlockSpec((1, T2, FB), lambda b, fb, p: (b, 0, NF + fb))
    spec_out = pl.BlockSpec((1, T2, FB), lambda b, fb, p: (b, 0, p * NF + fb))

    main_r, det_r = pl.pallas_call(
        _tc_body,
        grid=(B, NF, 2),
        in_specs=[spec_e, spec_o],
        out_specs=[spec_out, spec_out],
        out_shape=[
            jax.ShapeDtypeStruct((B, T2, 2 * F), jnp.float32),
            jax.ShapeDtypeStruct((B, T2, 2 * F), jnp.float32),
        ],
        scratch_shapes=[pltpu.VMEM((T2, FB), jnp.float32)],
    )(xr, xr)
    return main_r.reshape(B, T, F), det_r.reshape(B, T, F)
